# Initial kernel scaffold; baseline (speedup 1.0000x reference)
import jax
import jax.numpy as jnp
from jax.experimental import pallas as pl


def kernel(x, target):
    raise NotImplementedError("write your pallas kernel here")


# speedup vs baseline: 1.6194x; 1.6194x over previous
import functools
import math

import jax
import jax.numpy as jnp
from jax.experimental import pallas as pl

_SIZE = 100000
_PAD = 0
_SMOOTHING = 0.1
_CONF = 1.0 - _SMOOTHING
_EPS = _SMOOTHING / (_SIZE - 2)
_C1 = _SMOOTHING * math.log(_EPS) + _CONF * math.log(_CONF)

_RB = 8


def _loss_kernel(x_ref, tgt_ref, out_ref):
    i = pl.program_id(0)

    @pl.when(i == 0)
    def _init():
        out_ref[...] = jnp.zeros((1, 1), jnp.float32)

    xb = x_ref[...]
    tgt = tgt_ref[...]
    col = jax.lax.broadcasted_iota(jnp.int32, xb.shape, 1)
    x_t = jnp.sum(jnp.where(col == tgt, xb, 0.0), axis=1, keepdims=True)
    s = jnp.sum(xb, axis=1, keepdims=True)
    x0 = xb[:, 0:1]
    valid = (tgt != _PAD).astype(jnp.float32)
    per_row = _C1 - _EPS * s + _EPS * x0 + (_EPS - _CONF) * x_t
    out_ref[...] += jnp.sum(valid * per_row, axis=0, keepdims=True)


@functools.partial(jax.jit, static_argnames=())
def kernel(x, target):
    n = x.shape[0]
    tgt = target.astype(jnp.int32).reshape(n, 1)
    out = pl.pallas_call(
        _loss_kernel,
        grid=(n // _RB,),
        in_specs=[
            pl.BlockSpec((_RB, _SIZE), lambda i: (i, 0)),
            pl.BlockSpec((_RB, 1), lambda i: (i, 0)),
        ],
        out_specs=pl.BlockSpec((1, 1), lambda i: (0, 0)),
        out_shape=jax.ShapeDtypeStruct((1, 1), jnp.float32),
    )(x, tgt)
    return out[0, 0]


# speedup vs baseline: 1.8681x; 1.1536x over previous
import functools
import math

import jax
import jax.numpy as jnp
from jax.experimental import pallas as pl

_SIZE = 100000
_PAD = 0
_SMOOTHING = 0.1
_CONF = 1.0 - _SMOOTHING
_EPS = _SMOOTHING / (_SIZE - 2)
_C1 = _SMOOTHING * math.log(_EPS) + _CONF * math.log(_CONF)

_RB = 32


def _loss_kernel(x_ref, tgt_ref, out_ref):
    i = pl.program_id(0)

    @pl.when(i == 0)
    def _init():
        out_ref[...] = jnp.zeros((1, 1), jnp.float32)

    xb = x_ref[...]
    tgt = tgt_ref[...]
    col = jax.lax.broadcasted_iota(jnp.int32, xb.shape, 1)
    w = jnp.where(col == tgt, -_CONF, -_EPS)
    ws = jnp.sum(w * xb, axis=1, keepdims=True)
    x0 = xb[:, 0:1]
    valid = (tgt != _PAD).astype(jnp.float32)
    per_row = _C1 + ws + _EPS * x0
    out_ref[...] += jnp.sum(valid * per_row, axis=0, keepdims=True)


@functools.partial(jax.jit, static_argnames=())
def kernel(x, target):
    n = x.shape[0]
    tgt = target.astype(jnp.int32).reshape(n, 1)
    out = pl.pallas_call(
        _loss_kernel,
        grid=(n // _RB,),
        in_specs=[
            pl.BlockSpec((_RB, _SIZE), lambda i: (i, 0)),
            pl.BlockSpec((_RB, 1), lambda i: (i, 0)),
        ],
        out_specs=pl.BlockSpec((1, 1), lambda i: (0, 0)),
        out_shape=jax.ShapeDtypeStruct((1, 1), jnp.float32),
    )(x, tgt)
    return out[0, 0]
